# baseline (speedup 1.0000x reference)
import functools

import jax
import jax.numpy as jnp
from jax import lax
from jax.experimental import pallas as pl
from jax.experimental.pallas import tpu as pltpu
from jax.experimental.pallas import tpu_sc as plsc

N = 10000
E = 320000
EH = E // 2
D = 128

NC = 2
NS = 16
NW = NC * NS
EPW = EH // NW
CH = 40
NCH = EPW // CH
NB = 5
NJ = NCH // NB
NPAD = 10240
NPS = NPAD // NS

_mesh = plsc.VectorSubcoreMesh(core_axis_name="c", subcore_axis_name="s")


@functools.partial(
    pl.kernel,
    out_type=jax.ShapeDtypeStruct((EH, D), jnp.float32),
    mesh=_mesh,
    scratch_types=[
        pltpu.VMEM((EPW,), jnp.int32),
        pltpu.VMEM((NB, CH, D), jnp.float32),
        pltpu.SemaphoreType.DMA((NB,)),
        pltpu.SemaphoreType.DMA((NB,)),
    ],
)
def _gather_xj(x_hbm, src_hbm, xj_hbm, idx_all, rows_v, gsem, ssem):
    wid = lax.axis_index("s") * NC + lax.axis_index("c")
    base = wid * EPW
    pltpu.sync_copy(src_hbm.at[pl.ds(base, EPW)], idx_all)

    def gath(i, k):
        return pltpu.make_async_copy(
            x_hbm.at[idx_all.at[pl.ds(i * CH, CH)]], rows_v.at[k], gsem.at[k])

    def store(i, k):
        return pltpu.make_async_copy(
            rows_v.at[k], xj_hbm.at[pl.ds(base + i * CH, CH)], ssem.at[k])

    def body(j, carry):
        for k in range(NB):
            i = j * NB + k

            @pl.when(j >= 1)
            def _():
                store(i - NB, k).wait()

            gath(i, k).start()
        for k in range(NB):
            i = j * NB + k
            gath(i, k).wait()
            store(i, k).start()
        return carry

    lax.fori_loop(0, NJ, body, 0)
    for k in range(NB):
        store((NJ - 1) * NB + k, k).wait()


def _msg_body(emb_ref, xj_ref, w_ref, m_ref):
    rw = jnp.dot(emb_ref[...], w_ref[...], preferred_element_type=jnp.float32)
    m_ref[...] = rw * xj_ref[...]


_BE = 2000


def _msg(edge_emb, xj, l_weight):
    return pl.pallas_call(
        _msg_body,
        grid=(EH // _BE,),
        in_specs=[
            pl.BlockSpec((_BE, D), lambda i: (i, 0)),
            pl.BlockSpec((_BE, D), lambda i: (i, 0)),
            pl.BlockSpec((D, D), lambda i: (0, 0)),
        ],
        out_specs=pl.BlockSpec((_BE, D), lambda i: (i, 0)),
        out_shape=jax.ShapeDtypeStruct((EH, D), jnp.float32),
        compiler_params=pltpu.CompilerParams(
            dimension_semantics=("arbitrary",)),
    )(edge_emb, xj, l_weight)


@functools.partial(
    pl.kernel,
    out_type=(
        jax.ShapeDtypeStruct((NC * NPAD, D), jnp.float32),
        jax.ShapeDtypeStruct((NC * NPAD,), jnp.float32),
    ),
    mesh=_mesh,
    scratch_types=[
        pltpu.VMEM((NB, CH), jnp.int32),
        pltpu.VMEM((NB, CH, D), jnp.float32),
        pltpu.VMEM((CH,), jnp.float32),
        pltpu.VMEM_SHARED((NPAD, D), jnp.float32),
        pltpu.VMEM_SHARED((NPAD,), jnp.float32),
        pltpu.SemaphoreType.DMA((NB,)),
        pltpu.SemaphoreType.DMA((NB,)),
        pltpu.SemaphoreType.DMA((NB,)),
        pltpu.SemaphoreType.DMA((NB,)),
    ],
)
def _scatter_sums(m_hbm, dst3_hbm, za_hbm, zc1_hbm, ones1_hbm, pa_hbm, pc1_hbm,
                  idxb, m_v, ones1_v, acc_a, acc_c1, isem, msem, scsem, csem):
    c = lax.axis_index("c")
    s = lax.axis_index("s")
    wid = s * NC + c
    base = wid * EPW

    pltpu.sync_copy(za_hbm.at[pl.ds(s * NPS, NPS)], acc_a.at[pl.ds(s * NPS, NPS)])
    pltpu.sync_copy(zc1_hbm.at[pl.ds(s * NPS, NPS)], acc_c1.at[pl.ds(s * NPS, NPS)])
    pltpu.sync_copy(ones1_hbm, ones1_v)
    plsc.subcore_barrier()

    def iload(i, k):
        return pltpu.make_async_copy(
            dst3_hbm.at[wid, i], idxb.at[k], isem.at[k])

    def mload(i, k):
        return pltpu.make_async_copy(
            m_hbm.at[pl.ds(base + i * CH, CH)], m_v.at[k], msem.at[k])

    def scat(i, k):
        return pltpu.make_async_copy(
            m_v.at[k], acc_a.at[idxb.at[k]], scsem.at[k])

    def cscat(i, k):
        return pltpu.make_async_copy(
            ones1_v, acc_c1.at[idxb.at[k]], csem.at[k])

    def body(j, carry):
        for k in range(NB):
            i = j * NB + k

            @pl.when(j >= 1)
            def _():
                scat(i - NB, k).wait()
                cscat(i - NB, k).wait()

            iload(i, k).start()
            mload(i, k).start()
        for k in range(NB):
            i = j * NB + k
            iload(i, k).wait()
            mload(i, k).wait()
            pltpu.async_copy(m_v.at[k], acc_a.at[idxb.at[k]],
                             scsem.at[k], add=True)
            pltpu.async_copy(ones1_v, acc_c1.at[idxb.at[k]],
                             csem.at[k], add=True)
        return carry

    lax.fori_loop(0, NJ, body, 0)
    for k in range(NB):
        scat((NJ - 1) * NB + k, k).wait()
        cscat((NJ - 1) * NB + k, k).wait()
    plsc.subcore_barrier()

    out_row = c * NPAD + s * NPS
    pltpu.sync_copy(acc_a.at[pl.ds(s * NPS, NPS)], pa_hbm.at[pl.ds(out_row, NPS)])
    pltpu.sync_copy(acc_c1.at[pl.ds(s * NPS, NPS)], pc1_hbm.at[pl.ds(out_row, NPS)])


_BN = 1024


def _combine_body(pa0_ref, pa1_ref, pc0_ref, pc1_ref,
                  x_ref, root_ref, bias_ref, out_ref):
    ssum = pa0_ref[0] + pa0_ref[1] + pa1_ref[0] + pa1_ref[1]
    cnt = (pc0_ref[0, :] + pc0_ref[1, :]
           + pc1_ref[0, :] + pc1_ref[1, :]).reshape(_BN, 1)
    xr = jnp.dot(x_ref[...], root_ref[...], preferred_element_type=jnp.float32)
    out_ref[...] = ssum / jnp.maximum(cnt, 1.0) + xr + bias_ref[...]


def _combine(pa0, pa1, pc0, pc1, x, root, bias2d):
    return pl.pallas_call(
        _combine_body,
        grid=(NPAD // _BN,),
        in_specs=[
            pl.BlockSpec((NC, _BN, D), lambda i: (0, i, 0)),
            pl.BlockSpec((NC, _BN, D), lambda i: (0, i, 0)),
            pl.BlockSpec((NC, _BN), lambda i: (0, i)),
            pl.BlockSpec((NC, _BN), lambda i: (0, i)),
            pl.BlockSpec((_BN, D), lambda i: (i, 0)),
            pl.BlockSpec((D, D), lambda i: (0, 0)),
            pl.BlockSpec((1, D), lambda i: (0, 0)),
        ],
        out_specs=pl.BlockSpec((_BN, D), lambda i: (i, 0)),
        out_shape=jax.ShapeDtypeStruct((NPAD, D), jnp.float32),
        compiler_params=pltpu.CompilerParams(
            dimension_semantics=("arbitrary",)),
    )(pa0, pa1, pc0, pc1, x, root, bias2d)


def kernel(x, edge_index, edge_emb, l_weight, root, message_bias):
    src = edge_index[0]
    dst = edge_index[1]
    za = jnp.zeros((NPAD, D), jnp.float32)
    zc1 = jnp.zeros((NPAD,), jnp.float32)
    ones1 = jnp.ones((CH,), jnp.float32)

    src0, src1 = src[:EH], src[EH:]
    dst30 = dst[:EH].reshape(NW, NCH, CH)
    dst31 = dst[EH:].reshape(NW, NCH, CH)
    emb0, emb1 = edge_emb[:EH], edge_emb[EH:]

    xj0 = _gather_xj(x, src0)
    m0 = _msg(emb0, xj0, l_weight)
    xj1 = _gather_xj(x, src1)
    pa0, pc0 = _scatter_sums(m0, dst30, za, zc1, ones1)
    m1 = _msg(emb1, xj1, l_weight)
    pa1, pc1 = _scatter_sums(m1, dst31, za, zc1, ones1)

    xp = jnp.concatenate([x, jnp.zeros((NPAD - N, D), jnp.float32)], axis=0)
    out = _combine(pa0.reshape(NC, NPAD, D), pa1.reshape(NC, NPAD, D),
                   pc0.reshape(NC, NPAD), pc1.reshape(NC, NPAD),
                   xp, root, message_bias.reshape(1, D))
    return out[:N]


# speedup vs baseline: 1.1483x; 1.1483x over previous
import functools

import jax
import jax.numpy as jnp
from jax import lax
from jax.experimental import pallas as pl
from jax.experimental.pallas import tpu as pltpu
from jax.experimental.pallas import tpu_sc as plsc

N = 10000
E = 320000
D = 128

NC = 2
NS = 16
NW = NC * NS
EPW = E // NW
CH = 80
NCH = EPW // CH
NB = 5
NJ = NCH // NB
NPAD = 10240
NPS = NPAD // NS
CHS = 40
NCHS = EPW // CHS
NJS = NCHS // NB

_mesh = plsc.VectorSubcoreMesh(core_axis_name="c", subcore_axis_name="s")


@functools.partial(
    pl.kernel,
    out_type=jax.ShapeDtypeStruct((E, D), jnp.float32),
    mesh=_mesh,
    scratch_types=[
        pltpu.VMEM((EPW,), jnp.int32),
        pltpu.VMEM((NB, CH, D), jnp.float32),
        pltpu.SemaphoreType.DMA((NB,)),
        pltpu.SemaphoreType.DMA((NB,)),
    ],
)
def _gather_xj(x_hbm, src_hbm, xj_hbm, idx_all, rows_v, gsem, ssem):
    wid = lax.axis_index("s") * NC + lax.axis_index("c")
    base = wid * EPW
    pltpu.sync_copy(src_hbm.at[pl.ds(base, EPW)], idx_all)

    def gath(i, k):
        return pltpu.make_async_copy(
            x_hbm.at[idx_all.at[pl.ds(i * CH, CH)]], rows_v.at[k], gsem.at[k])

    def store(i, k):
        return pltpu.make_async_copy(
            rows_v.at[k], xj_hbm.at[pl.ds(base + i * CH, CH)], ssem.at[k])

    def body(j, carry):
        for k in range(NB):
            i = j * NB + k

            @pl.when(j >= 1)
            def _():
                store(i - NB, k).wait()

            gath(i, k).start()
        for k in range(NB):
            i = j * NB + k
            gath(i, k).wait()
            store(i, k).start()
        return carry

    lax.fori_loop(0, NJ, body, 0)
    for k in range(NB):
        store((NJ - 1) * NB + k, k).wait()


def _msg_body(emb_ref, xj_ref, w_ref, m_ref):
    rw = jnp.dot(emb_ref[...], w_ref[...], preferred_element_type=jnp.float32)
    m_ref[...] = rw * xj_ref[...]


_BE = 2000


def _msg(edge_emb, xj, l_weight):
    return pl.pallas_call(
        _msg_body,
        grid=(E // _BE,),
        in_specs=[
            pl.BlockSpec((_BE, D), lambda i: (i, 0)),
            pl.BlockSpec((_BE, D), lambda i: (i, 0)),
            pl.BlockSpec((D, D), lambda i: (0, 0)),
        ],
        out_specs=pl.BlockSpec((_BE, D), lambda i: (i, 0)),
        out_shape=jax.ShapeDtypeStruct((E, D), jnp.float32),
        compiler_params=pltpu.CompilerParams(
            dimension_semantics=("arbitrary",)),
    )(edge_emb, xj, l_weight)


@functools.partial(
    pl.kernel,
    out_type=jax.ShapeDtypeStruct((NC * NPAD, D), jnp.float32),
    mesh=_mesh,
    scratch_types=[
        pltpu.VMEM((NB, CHS), jnp.int32),
        pltpu.VMEM((NB, CHS, D), jnp.float32),
        pltpu.VMEM_SHARED((NPAD, D), jnp.float32),
        pltpu.SemaphoreType.DMA((NB,)),
        pltpu.SemaphoreType.DMA((NB,)),
        pltpu.SemaphoreType.DMA((NB,)),
    ],
)
def _scatter_sums(m_hbm, dst3_hbm, za_hbm, pa_hbm,
                  idxb, m_v, acc_a, isem, msem, scsem):
    c = lax.axis_index("c")
    s = lax.axis_index("s")
    wid = s * NC + c
    base = wid * EPW

    pltpu.sync_copy(za_hbm.at[pl.ds(s * NPS, NPS)], acc_a.at[pl.ds(s * NPS, NPS)])
    plsc.subcore_barrier()

    def iload(i, k):
        return pltpu.make_async_copy(
            dst3_hbm.at[wid, i], idxb.at[k], isem.at[k])

    def mload(i, k):
        return pltpu.make_async_copy(
            m_hbm.at[pl.ds(base + i * CHS, CHS)], m_v.at[k], msem.at[k])

    def scat(i, k):
        return pltpu.make_async_copy(
            m_v.at[k], acc_a.at[idxb.at[k]], scsem.at[k])

    def body(j, carry):
        for k in range(NB):
            i = j * NB + k

            @pl.when(j >= 1)
            def _():
                scat(i - NB, k).wait()

            iload(i, k).start()
            mload(i, k).start()
        for k in range(NB):
            i = j * NB + k
            iload(i, k).wait()
            mload(i, k).wait()
            pltpu.async_copy(m_v.at[k], acc_a.at[idxb.at[k]],
                             scsem.at[k], add=True)
        return carry

    lax.fori_loop(0, NJS, body, 0)
    for k in range(NB):
        scat((NJS - 1) * NB + k, k).wait()
    plsc.subcore_barrier()

    out_row = c * NPAD + s * NPS
    pltpu.sync_copy(acc_a.at[pl.ds(s * NPS, NPS)], pa_hbm.at[pl.ds(out_row, NPS)])


@functools.partial(
    pl.kernel,
    out_type=jax.ShapeDtypeStruct((NC * NPAD, D), jnp.float32),
    mesh=_mesh,
    scratch_types=[
        pltpu.VMEM((NB, CHS), jnp.int32),
        pltpu.VMEM((CHS, D), jnp.float32),
        pltpu.VMEM_SHARED((NPAD, D), jnp.float32),
        pltpu.SemaphoreType.DMA((NB,)),
        pltpu.SemaphoreType.DMA((NB,)),
    ],
)
def _scatter_counts(dst3_hbm, za_hbm, ones_hbm, pc_hbm,
                    idxb, ones_v, acc_c, isem, csem):
    c = lax.axis_index("c")
    s = lax.axis_index("s")
    wid = s * NC + c

    pltpu.sync_copy(za_hbm.at[pl.ds(s * NPS, NPS)], acc_c.at[pl.ds(s * NPS, NPS)])
    pltpu.sync_copy(ones_hbm, ones_v)
    plsc.subcore_barrier()

    def iload(i, k):
        return pltpu.make_async_copy(
            dst3_hbm.at[wid, i], idxb.at[k], isem.at[k])

    def cscat(i, k):
        return pltpu.make_async_copy(
            ones_v, acc_c.at[idxb.at[k]], csem.at[k])

    def body(j, carry):
        for k in range(NB):
            i = j * NB + k

            @pl.when(j >= 1)
            def _():
                cscat(i - NB, k).wait()

            iload(i, k).start()
        for k in range(NB):
            i = j * NB + k
            iload(i, k).wait()
            pltpu.async_copy(ones_v, acc_c.at[idxb.at[k]],
                             csem.at[k], add=True)
        return carry

    lax.fori_loop(0, NJS, body, 0)
    for k in range(NB):
        cscat((NJS - 1) * NB + k, k).wait()
    plsc.subcore_barrier()

    out_row = c * NPAD + s * NPS
    pltpu.sync_copy(acc_c.at[pl.ds(s * NPS, NPS)], pc_hbm.at[pl.ds(out_row, NPS)])


_BN = 1024


def _combine_body(pa_ref, pc_ref, x_ref, root_ref, bias_ref, out_ref):
    ssum = pa_ref[0] + pa_ref[1]
    cnt = pc_ref[0][:, 0:1] + pc_ref[1][:, 0:1]
    xr = jnp.dot(x_ref[...], root_ref[...], preferred_element_type=jnp.float32)
    out_ref[...] = ssum / jnp.maximum(cnt, 1.0) + xr + bias_ref[...]


def _combine(pa, pc, x, root, bias2d):
    return pl.pallas_call(
        _combine_body,
        grid=(NPAD // _BN,),
        in_specs=[
            pl.BlockSpec((NC, _BN, D), lambda i: (0, i, 0)),
            pl.BlockSpec((NC, _BN, D), lambda i: (0, i, 0)),
            pl.BlockSpec((_BN, D), lambda i: (i, 0)),
            pl.BlockSpec((D, D), lambda i: (0, 0)),
            pl.BlockSpec((1, D), lambda i: (0, 0)),
        ],
        out_specs=pl.BlockSpec((_BN, D), lambda i: (i, 0)),
        out_shape=jax.ShapeDtypeStruct((NPAD, D), jnp.float32),
        compiler_params=pltpu.CompilerParams(
            dimension_semantics=("arbitrary",)),
    )(pa, pc, x, root, bias2d)


def kernel(x, edge_index, edge_emb, l_weight, root, message_bias):
    src = edge_index[0]
    dst3 = edge_index[1].reshape(NW, NCHS, CHS)
    za = jnp.zeros((NPAD, D), jnp.float32)
    ones = jnp.ones((CHS, D), jnp.float32)

    xj = _gather_xj(x, src)
    m = _msg(edge_emb, xj, l_weight)
    pa = _scatter_sums(m, dst3, za)
    pc = _scatter_counts(dst3, za, ones)
    xp = jnp.concatenate([x, jnp.zeros((NPAD - N, D), jnp.float32)], axis=0)
    out = _combine(pa.reshape(NC, NPAD, D), pc.reshape(NC, NPAD, D),
                   xp, root, message_bias.reshape(1, D))
    return out[:N]
